# Initial kernel scaffold; baseline (speedup 1.0000x reference)
import jax
import jax.numpy as jnp
from jax.experimental import pallas as pl


def kernel(x, edge_index, edge_attr, W1n, b1n, W1s, b1s, bn1_g, bn1_b, W2n, b2n, W2s, b2s, bn2_g, bn2_b):
    raise NotImplementedError("write your pallas kernel here")


# speedup vs baseline: 2.7122x; 2.7122x over previous
import jax
import jax.numpy as jnp
from jax import lax
from jax.experimental import pallas as pl
from jax.experimental.pallas import tpu as pltpu
from jax.experimental.pallas import tpu_sc as plsc

_N = 10000
_NPAD = 10240
_NE = 320000
_D = 128
_EAW = 32
_CNT_COL = 16
_EPSBN = 1e-5

_NC = 2
_NS = 16
_NW = _NC * _NS
_CH = 128
_NCHUNK = 79
_EPW = _CH * _NCHUNK
_NE_PAD = _EPW * _NW
_RPW = _NPAD // _NS


def _zero_rows(buf, nrows):
  def zero_row(i, _):
    for l in range(_D // 16):
      buf[i, pl.ds(l * 16, 16)] = jnp.zeros((16,), jnp.float32)
    return 0
  lax.fori_loop(0, nrows, zero_row, 0)


def _sc_gather_segsum():
  mesh = plsc.VectorSubcoreMesh(
      core_axis_name="c", subcore_axis_name="s", num_cores=_NC,
      num_subcores=_NS)

  def body(tab, src_h, dst_h, out_s, srcv, dstv, rowsv, acc_s, sem):
    c = lax.axis_index("c")
    s = lax.axis_index("s")
    base = (s * _NC + c) * _EPW

    _zero_rows(rowsv, _CH)
    for k in range(_RPW // _CH):
      pltpu.sync_copy(rowsv, acc_s.at[pl.ds(s * _RPW + k * _CH, _CH)])
    plsc.subcore_barrier()

    def chunk(j, _):
      off = base + j * _CH
      pltpu.sync_copy(src_h.at[pl.ds(off, _CH)], srcv)
      pltpu.sync_copy(dst_h.at[pl.ds(off, _CH)], dstv)
      pltpu.async_copy(tab.at[srcv], rowsv, sem).wait()
      pltpu.sync_copy(rowsv, acc_s.at[dstv], add=True)
      return 0
    lax.fori_loop(0, _NCHUNK, chunk, 0)
    plsc.subcore_barrier()

    r0 = s * _RPW
    for k in range(_RPW // _CH):
      pltpu.sync_copy(acc_s.at[pl.ds(r0 + k * _CH, _CH)], rowsv)
      pltpu.sync_copy(rowsv, out_s.at[c, pl.ds(r0 + k * _CH, _CH)])

  return pl.kernel(
      body,
      out_type=[jax.ShapeDtypeStruct((_NC, _NPAD, _D), jnp.float32)],
      mesh=mesh,
      scratch_types=[
          pltpu.VMEM((_CH,), jnp.int32),
          pltpu.VMEM((_CH,), jnp.int32),
          pltpu.VMEM((_CH, _D), jnp.float32),
          pltpu.VMEM_SHARED((_NPAD, _D), jnp.float32),
          pltpu.SemaphoreType.DMA,
      ])


def _sc_edge_segsum():
  mesh = plsc.VectorSubcoreMesh(
      core_axis_name="c", subcore_axis_name="s", num_cores=_NC,
      num_subcores=_NS)

  def body(ea_h, dst_h, out_e, dstv, eav, rowsv, acc_e):
    c = lax.axis_index("c")
    s = lax.axis_index("s")
    base = (s * _NC + c) * _EPW

    _zero_rows(rowsv, _CH)
    for k in range(_RPW // _CH):
      pltpu.sync_copy(rowsv, acc_e.at[pl.ds(s * _RPW + k * _CH, _CH)])
    plsc.subcore_barrier()

    def chunk(j, _):
      off = base + j * _CH
      pltpu.sync_copy(dst_h.at[pl.ds(off, _CH)], dstv)
      pltpu.sync_copy(ea_h.at[pl.ds(off, _CH)], eav)

      def expand(i, _):
        for l in range(_EAW // 16):
          rowsv[i, pl.ds(l * 16, 16)] = eav[i, pl.ds(l * 16, 16)]
        return 0
      lax.fori_loop(0, _CH, expand, 0)
      pltpu.sync_copy(rowsv, acc_e.at[dstv], add=True)
      return 0
    lax.fori_loop(0, _NCHUNK, chunk, 0)
    plsc.subcore_barrier()

    r0 = s * _RPW
    for k in range(_RPW // _CH):
      pltpu.sync_copy(acc_e.at[pl.ds(r0 + k * _CH, _CH)], rowsv)
      pltpu.sync_copy(rowsv, out_e.at[c, pl.ds(r0 + k * _CH, _CH)])

  return pl.kernel(
      body,
      out_type=[jax.ShapeDtypeStruct((_NC, _NPAD, _D), jnp.float32)],
      mesh=mesh,
      scratch_types=[
          pltpu.VMEM((_CH,), jnp.int32),
          pltpu.VMEM((_CH, _EAW), jnp.float32),
          pltpu.VMEM((_CH, _D), jnp.float32),
          pltpu.VMEM_SHARED((_NPAD, _D), jnp.float32),
      ])


def _tc_body(x_ref, s_ref, e_ref, wsT_ref, wnnT_ref, wneT_ref, bs_ref,
             g_ref, b_ref, o_ref):
  hi = jax.lax.Precision.HIGHEST
  S = s_ref[0] + s_ref[1]
  E = e_ref[0] + e_ref[1]
  counts = E[:, _CNT_COL:_CNT_COL + 1]
  denom = 1.0 / jnp.maximum(counts, 1.0)
  aggr = (jnp.dot(S, wnnT_ref[...], precision=hi)
          + jnp.dot(E, wneT_ref[...], precision=hi)) * denom
  xs = jnp.dot(x_ref[...], wsT_ref[...], precision=hi) + bs_ref[...]
  h = jnp.maximum(xs + aggr, 0.0)
  o_ref[...] = jnp.maximum(h * g_ref[...] + b_ref[...], 0.0)


def _tc_layer(xp, Sp, Ep, wsT, wnnT, wneT, bs, g, b):
  br = 2048
  grid = (_NPAD // br,)
  return pl.pallas_call(
      _tc_body,
      grid=grid,
      in_specs=[
          pl.BlockSpec((br, _D), lambda i: (i, 0)),
          pl.BlockSpec((_NC, br, _D), lambda i: (0, i, 0)),
          pl.BlockSpec((_NC, br, _D), lambda i: (0, i, 0)),
          pl.BlockSpec((_D, _D), lambda i: (0, 0)),
          pl.BlockSpec((_D, _D), lambda i: (0, 0)),
          pl.BlockSpec((_D, _D), lambda i: (0, 0)),
          pl.BlockSpec((1, _D), lambda i: (0, 0)),
          pl.BlockSpec((1, _D), lambda i: (0, 0)),
          pl.BlockSpec((1, _D), lambda i: (0, 0)),
      ],
      out_specs=pl.BlockSpec((br, _D), lambda i: (i, 0)),
      out_shape=jax.ShapeDtypeStruct((_NPAD, _D), jnp.float32),
  )(xp, Sp, Ep, wsT, wnnT, wneT, bs, g, b)


def _prep_neighbor_weights(Wn, bn_):
  wnnT = Wn.T[:_D]
  wneT = jnp.zeros((_D, _D), jnp.float32)
  wneT = wneT.at[:16].set(Wn.T[_D:_D + 16]).at[_CNT_COL].set(bn_)
  return wnnT, wneT


def kernel(x, edge_index, edge_attr, W1n, b1n, W1s, b1s, bn1_g, bn1_b,
           W2n, b2n, W2s, b2s, bn2_g, bn2_b):
  f32 = jnp.float32
  src = edge_index[0].astype(jnp.int32)
  dst = edge_index[1].astype(jnp.int32)
  padn = _NE_PAD - _NE
  srcp = jnp.concatenate([src, jnp.zeros((padn,), jnp.int32)])
  dstp = jnp.concatenate([dst, jnp.full((padn,), _N, jnp.int32)])
  eap = jnp.zeros((_NE_PAD, _EAW), f32)
  eap = eap.at[:_NE, :16].set(edge_attr.astype(f32)).at[:_NE, _CNT_COL].set(1.0)
  xp = jnp.zeros((_NPAD, _D), f32).at[:_N].set(x.astype(f32))

  sc_gather = _sc_gather_segsum()
  (Ep,) = _sc_edge_segsum()(eap, dstp)
  (S1p,) = sc_gather(xp, srcp, dstp)

  bnscale = 1.0 / jnp.sqrt(1.0 + _EPSBN)
  w1nnT, w1neT = _prep_neighbor_weights(W1n, b1n)
  h = _tc_layer(xp, S1p, Ep, W1s.T, w1nnT, w1neT, b1s[None],
                (bn1_g * bnscale)[None], bn1_b[None])

  (S2p,) = sc_gather(h, srcp, dstp)
  w2nnT, w2neT = _prep_neighbor_weights(W2n, b2n)
  out = _tc_layer(h, S2p, Ep, W2s.T, w2nnT, w2neT, b2s[None],
                  (bn2_g * bnscale)[None], bn2_b[None])
  return out[:_N]
